# Initial kernel scaffold; baseline (speedup 1.0000x reference)
import jax
import jax.numpy as jnp
from jax.experimental import pallas as pl


def kernel(x, edge_index, node_feat_mask, edge_mask, W1, W2, pred_label):
    raise NotImplementedError("write your pallas kernel here")


# speedup vs baseline: 5.0178x; 5.0178x over previous
import jax
import jax.numpy as jnp
from jax import lax
from jax.experimental import pallas as pl
from jax.experimental.pallas import tpu as pltpu
from jax.experimental.pallas import tpu_sc as plsc

N = 10000
E = 160000
F = 256
H = 512
C = 128

NB = 25
BN = N // NB
BE = E // NB
RPT = N // 16


def _t1_body(x_ref, nf_ref, em_ref, h_ref, ew_ref):
    nfs = jax.nn.sigmoid(nf_ref[0])
    hb = x_ref[...] * nfs[None, :]
    h_ref[0] = hb[:, :128]
    h_ref[1] = hb[:, 128:]
    ew_ref[...] = jax.nn.sigmoid(em_ref[...])


def _stage1(x, node_feat_mask, edge_mask2d):
    return pl.pallas_call(
        _t1_body,
        grid=(NB,),
        in_specs=[
            pl.BlockSpec((BN, F), lambda i: (i, 0)),
            pl.BlockSpec((1, F), lambda i: (0, 0)),
            pl.BlockSpec((1, 1, BE), lambda i: (i, 0, 0)),
        ],
        out_specs=[
            pl.BlockSpec((2, BN, 128), lambda i: (0, i, 0)),
            pl.BlockSpec((1, 1, BE), lambda i: (i, 0, 0)),
        ],
        out_shape=[
            jax.ShapeDtypeStruct((2, N, 128), jnp.float32),
            jax.ShapeDtypeStruct((NB, 1, BE), jnp.float32),
        ],
    )(x, node_feat_mask, edge_mask2d)


_DN = lax.GatherDimensionNumbers(offset_dims=(), collapsed_slice_dims=(0,),
                                 start_index_map=(0,))


def _lane_splat(vec16, i):
    idx = jnp.full((16, 1), i, jnp.int32)
    return lax.gather(vec16, idx, _DN, slice_sizes=(1,),
                      mode=lax.GatherScatterMode.PROMISE_IN_BOUNDS)


ZROWS = 624
ZLAST = N - 15 * ZROWS


def _sc_segsum(tables, src, dst, ew, zeros, *, ept, blk,
               split_edges_by_core):
    nblk = ept // blk
    nt = len(tables)
    mesh = plsc.VectorSubcoreMesh(core_axis_name="c", subcore_axis_name="s")

    def body(*refs):
        tabs = refs[:nt]
        src_h, dst_h, ew_h, z_h, o0, o1 = refs[nt:nt + 6]
        srcv, ewv, dst_blk, rowsv, sem, agg = refs[nt + 6:]
        c = lax.axis_index("c")
        s = lax.axis_index("s")
        tile = (c * 16 + s) if split_edges_by_core else s
        ebase = pl.multiple_of(tile * ept, 8)
        pltpu.sync_copy(src_h.at[pl.ds(ebase, ept)], srcv)
        pltpu.sync_copy(ew_h.at[pl.ds(ebase, ept)], ewv)
        zoff = pl.multiple_of(s * ZROWS, 8)

        @pl.when(s < 15)
        def _():
            pltpu.sync_copy(z_h.at[pl.ds(0, ZROWS)],
                            agg.at[pl.ds(zoff, ZROWS)])

        @pl.when(s == 15)
        def _():
            pltpu.sync_copy(z_h, agg.at[pl.ds(15 * ZROWS, ZLAST)])

        plsc.subcore_barrier()

        def blockfn(j, carry):
            off = pl.multiple_of(j * blk, 8)
            idx = srcv.at[pl.ds(off, blk)]
            pltpu.sync_copy(dst_h.at[pl.ds(ebase + off, blk)], dst_blk)
            if nt == 2:
                @pl.when(c == 0)
                def _():
                    pltpu.async_copy(tabs[0].at[idx], rowsv, sem).wait()

                @pl.when(c == 1)
                def _():
                    pltpu.async_copy(tabs[1].at[idx], rowsv, sem).wait()
            else:
                pltpu.async_copy(tabs[0].at[idx], rowsv, sem).wait()
            for g in range(blk // 16):
                ew16 = ewv[pl.ds(pl.multiple_of(j * blk + g * 16, 8), 16)]
                for i in range(16):
                    w = _lane_splat(ew16, i)
                    e = g * 16 + i
                    for k in range(8):
                        sl = pl.ds(k * 16, 16)
                        rowsv[e, sl] = rowsv[e, sl] * w
            pltpu.sync_copy(rowsv, agg.at[dst_blk], add=True)
            return carry

        lax.fori_loop(0, nblk, blockfn, 0)
        plsc.subcore_barrier()

        def writeback(dst_ref):
            @pl.when(s < 15)
            def _():
                pltpu.sync_copy(agg.at[pl.ds(zoff, ZROWS)],
                                dst_ref.at[pl.ds(zoff, ZROWS)])

            @pl.when(s == 15)
            def _():
                pltpu.sync_copy(agg.at[pl.ds(15 * ZROWS, ZLAST)],
                                dst_ref.at[pl.ds(15 * ZROWS, ZLAST)])

        @pl.when(c == 0)
        def _():
            writeback(o0)

        @pl.when(c == 1)
        def _():
            writeback(o1)

    kern = pl.kernel(
        body,
        out_type=(jax.ShapeDtypeStruct((N, 128), jnp.float32),
                  jax.ShapeDtypeStruct((N, 128), jnp.float32)),
        mesh=mesh,
        scratch_types=[
            pltpu.VMEM((ept,), jnp.int32),
            pltpu.VMEM((ept,), jnp.float32),
            pltpu.VMEM((blk,), jnp.int32),
            pltpu.VMEM((blk, 128), jnp.float32),
            pltpu.SemaphoreType.DMA,
            pltpu.VMEM_SHARED((N, 128), jnp.float32),
        ],
    )
    return kern(*tables, src, dst, ew, zeros)


def _t2_body(a0_ref, a1_ref, w1_ref, w2_ref, y_ref):
    z = (jnp.dot(a0_ref[...], w1_ref[:128, :],
                 preferred_element_type=jnp.float32)
         + jnp.dot(a1_ref[...], w1_ref[128:, :],
                   preferred_element_type=jnp.float32))
    h1 = jnp.maximum(z, 0.0)
    y_ref[...] = jnp.dot(h1, w2_ref[...], preferred_element_type=jnp.float32)


def _stage2(a0, a1, W1, W2):
    return pl.pallas_call(
        _t2_body,
        grid=(NB,),
        in_specs=[
            pl.BlockSpec((BN, 128), lambda i: (i, 0)),
            pl.BlockSpec((BN, 128), lambda i: (i, 0)),
            pl.BlockSpec((F, H), lambda i: (0, 0)),
            pl.BlockSpec((H, C), lambda i: (0, 0)),
        ],
        out_specs=pl.BlockSpec((BN, C), lambda i: (i, 0)),
        out_shape=jax.ShapeDtypeStruct((N, C), jnp.float32),
    )(a0, a1, W1, W2)


def _t3_body(p0_ref, p1_ref, pred_ref, ew_ref, nf_ref, acc_ref):
    i = pl.program_id(0)

    @pl.when(i == 0)
    def _():
        nfm = jax.nn.sigmoid(nf_ref[...])
        ent2 = (-nfm * jnp.log(nfm + 1e-15)
                - (1.0 - nfm) * jnp.log(1.0 - nfm + 1e-15))
        acc_ref[0, 0] = 0.5 * jnp.mean(nfm) + 0.2 * jnp.mean(ent2)

    logits = p0_ref[...] + p1_ref[...]
    probs = jax.nn.sigmoid(logits)
    eps = 1e-12
    p = jnp.clip(probs, eps, 1.0 - eps)
    pred = pred_ref[...]
    bce = jnp.sum(pred * jnp.log(p) + (1.0 - pred) * jnp.log(1.0 - p))
    m = ew_ref[...]
    s_ew = jnp.sum(m)
    ent = -m * jnp.log(m + 1e-15) - (1.0 - m) * jnp.log(1.0 - m + 1e-15)
    s_ent = jnp.sum(ent)
    part = (-bce / (N * C)) + (0.01 / E) * s_ew + (0.5 / E) * s_ent
    acc_ref[0, 0] = acc_ref[0, 0] + part


def _stage3(p0, p1, pred_label, ew2d, node_feat_mask):
    return pl.pallas_call(
        _t3_body,
        grid=(NB,),
        in_specs=[
            pl.BlockSpec((BN, C), lambda i: (i, 0)),
            pl.BlockSpec((BN, C), lambda i: (i, 0)),
            pl.BlockSpec((BN, C), lambda i: (i, 0)),
            pl.BlockSpec((1, 1, BE), lambda i: (i, 0, 0)),
            pl.BlockSpec((1, F), lambda i: (0, 0)),
        ],
        out_specs=pl.BlockSpec((1, 1), lambda i: (0, 0),
                               memory_space=pltpu.MemorySpace.SMEM),
        out_shape=jax.ShapeDtypeStruct((1, 1), jnp.float32),
    )(p0, p1, pred_label, ew2d, node_feat_mask)


def kernel(x, edge_index, node_feat_mask, edge_mask, W1, W2, pred_label):
    src = edge_index[0]
    dst = edge_index[1]
    em2d = edge_mask.reshape(NB, 1, BE)

    h, ew2d = _stage1(x, node_feat_mask, em2d)
    ew = ew2d.reshape(E)
    zeros = jnp.zeros((ZLAST, 128), jnp.float32)

    a0, a1 = _sc_segsum(
        (h[0], h[1]), src, dst, ew, zeros,
        ept=E // 16, blk=80, split_edges_by_core=False)

    y = _stage2(a0, a1, W1, W2)

    p0, p1 = _sc_segsum(
        (y,), src, dst, ew, zeros,
        ept=E // 32, blk=40, split_edges_by_core=True)

    acc = _stage3(p0, p1, pred_label, ew2d, node_feat_mask)
    return acc[0, 0]


# speedup vs baseline: 7.2492x; 1.4447x over previous
import jax
import jax.numpy as jnp
from jax import lax
from jax.experimental import pallas as pl
from jax.experimental.pallas import tpu as pltpu
from jax.experimental.pallas import tpu_sc as plsc

N = 10000
E = 160000
F = 256
H = 512
C = 128

NB = 25
BN = N // NB
BE = E // NB

BLK = 80

_DN = lax.GatherDimensionNumbers(offset_dims=(), collapsed_slice_dims=(0,),
                                 start_index_map=(0,))


def _lane_splat(vec16, i):
    idx = jnp.full((16, 1), i, jnp.int32)
    return lax.gather(vec16, idx, _DN, slice_sizes=(1,),
                      mode=lax.GatherScatterMode.PROMISE_IN_BOUNDS)


ZROWS = 624
ZLAST = N - 15 * ZROWS


def _sc_segsum(tables, src, dst, em, zeros, *, split_edges_by_core):
    nt = len(tables)
    mesh = plsc.VectorSubcoreMesh(core_axis_name="c", subcore_axis_name="s")
    if split_edges_by_core:
        stage_len = 5120
    else:
        stage_len = 10080

    def body(*refs):
        tabs = refs[:nt]
        src_h, dst_h, em_h, z_h, o0, o1 = refs[nt:nt + 6]
        (srcv, emv, rows_a, rows_b, dst_a, dst_b,
         gsem_a, gsem_b, dsem_a, dsem_b, ssem_a, ssem_b, agg) = refs[nt + 6:]
        c = lax.axis_index("c")
        s = lax.axis_index("s")
        if split_edges_by_core:
            ebase = c * 80000 + jnp.where(s < 12, s * 4960,
                                          59520 + (s - 12) * 5120)
            npair = jnp.where(s < 12, 31, 32)
        else:
            ebase = jnp.where(s < 8, s * 9920, 79360 + (s - 8) * 10080)
            npair = jnp.where(s < 8, 62, 63)
        ebase = pl.multiple_of(ebase, 8)
        pltpu.sync_copy(src_h.at[pl.ds(ebase, stage_len)], srcv)
        pltpu.sync_copy(em_h.at[pl.ds(ebase, stage_len)], emv)
        zoff = pl.multiple_of(s * ZROWS, 8)

        @pl.when(s < 15)
        def _():
            pltpu.sync_copy(z_h.at[pl.ds(0, ZROWS)],
                            agg.at[pl.ds(zoff, ZROWS)])

        @pl.when(s == 15)
        def _():
            pltpu.sync_copy(z_h, agg.at[pl.ds(15 * ZROWS, ZLAST)])

        plsc.subcore_barrier()

        def scale(rows, boff):
            for g in range(BLK // 16):
                em16 = emv[pl.ds(pl.multiple_of(boff + g * 16, 8), 16)]
                ew16 = 1.0 / (1.0 + jnp.exp(-em16))
                for i in range(16):
                    w = _lane_splat(ew16, i)
                    e = g * 16 + i
                    for k in range(8):
                        sl = pl.ds(k * 16, 16)
                        rows[e, sl] = rows[e, sl] * w

        def make_pairfn(tab):
            def pairfn(jj, carry):
                off0 = pl.multiple_of(jj * (2 * BLK), 8)
                off1 = pl.multiple_of(off0 + BLK, 8)
                g_a = pltpu.async_copy(tab.at[srcv.at[pl.ds(off0, BLK)]],
                                       rows_a, gsem_a)
                g_b = pltpu.async_copy(tab.at[srcv.at[pl.ds(off1, BLK)]],
                                       rows_b, gsem_b)
                d_a = pltpu.async_copy(dst_h.at[pl.ds(ebase + off0, BLK)],
                                       dst_a, dsem_a)
                d_b = pltpu.async_copy(dst_h.at[pl.ds(ebase + off1, BLK)],
                                       dst_b, dsem_b)
                g_a.wait()
                scale(rows_a, off0)
                d_a.wait()
                s_a = pltpu.async_copy(rows_a, agg.at[dst_a], ssem_a,
                                       add=True)
                g_b.wait()
                scale(rows_b, off1)
                d_b.wait()
                s_b = pltpu.async_copy(rows_b, agg.at[dst_b], ssem_b,
                                       add=True)
                s_a.wait()
                s_b.wait()
                return carry
            return pairfn

        if nt == 2:
            @pl.when(c == 0)
            def _():
                lax.fori_loop(0, npair, make_pairfn(tabs[0]), 0)

            @pl.when(c == 1)
            def _():
                lax.fori_loop(0, npair, make_pairfn(tabs[1]), 0)
        else:
            lax.fori_loop(0, npair, make_pairfn(tabs[0]), 0)

        plsc.subcore_barrier()

        def writeback(dst_ref):
            @pl.when(s < 15)
            def _():
                pltpu.sync_copy(agg.at[pl.ds(zoff, ZROWS)],
                                dst_ref.at[pl.ds(zoff, ZROWS)])

            @pl.when(s == 15)
            def _():
                pltpu.sync_copy(agg.at[pl.ds(15 * ZROWS, ZLAST)],
                                dst_ref.at[pl.ds(15 * ZROWS, ZLAST)])

        @pl.when(c == 0)
        def _():
            writeback(o0)

        @pl.when(c == 1)
        def _():
            writeback(o1)

    kern = pl.kernel(
        body,
        out_type=(jax.ShapeDtypeStruct((N, 128), jnp.float32),
                  jax.ShapeDtypeStruct((N, 128), jnp.float32)),
        mesh=mesh,
        scratch_types=[
            pltpu.VMEM((stage_len,), jnp.int32),
            pltpu.VMEM((stage_len,), jnp.float32),
            pltpu.VMEM((BLK, 128), jnp.float32),
            pltpu.VMEM((BLK, 128), jnp.float32),
            pltpu.VMEM((BLK,), jnp.int32),
            pltpu.VMEM((BLK,), jnp.int32),
            pltpu.SemaphoreType.DMA,
            pltpu.SemaphoreType.DMA,
            pltpu.SemaphoreType.DMA,
            pltpu.SemaphoreType.DMA,
            pltpu.SemaphoreType.DMA,
            pltpu.SemaphoreType.DMA,
            pltpu.VMEM_SHARED((N, 128), jnp.float32),
        ],
    )
    return kern(*tables, src, dst, em, zeros)


def _t2_body(a0_ref, a1_ref, nf_ref, w1_ref, w2_ref, y_ref):
    nfs = jax.nn.sigmoid(nf_ref[0])
    a0 = a0_ref[...] * nfs[None, :128]
    a1 = a1_ref[...] * nfs[None, 128:]
    z = (jnp.dot(a0, w1_ref[:128, :], preferred_element_type=jnp.float32)
         + jnp.dot(a1, w1_ref[128:, :], preferred_element_type=jnp.float32))
    h1 = jnp.maximum(z, 0.0)
    y_ref[...] = jnp.dot(h1, w2_ref[...], preferred_element_type=jnp.float32)


def _stage2(a0, a1, node_feat_mask, W1, W2):
    return pl.pallas_call(
        _t2_body,
        grid=(NB,),
        in_specs=[
            pl.BlockSpec((BN, 128), lambda i: (i, 0)),
            pl.BlockSpec((BN, 128), lambda i: (i, 0)),
            pl.BlockSpec((1, F), lambda i: (0, 0)),
            pl.BlockSpec((F, H), lambda i: (0, 0)),
            pl.BlockSpec((H, C), lambda i: (0, 0)),
        ],
        out_specs=pl.BlockSpec((BN, C), lambda i: (i, 0)),
        out_shape=jax.ShapeDtypeStruct((N, C), jnp.float32),
    )(a0, a1, node_feat_mask, W1, W2)


def _t3_body(p0_ref, p1_ref, pred_ref, em_ref, nf_ref, acc_ref):
    i = pl.program_id(0)

    @pl.when(i == 0)
    def _():
        nfm = jax.nn.sigmoid(nf_ref[...])
        ent2 = (-nfm * jnp.log(nfm + 1e-15)
                - (1.0 - nfm) * jnp.log(1.0 - nfm + 1e-15))
        acc_ref[0, 0] = 0.5 * jnp.mean(nfm) + 0.2 * jnp.mean(ent2)

    logits = p0_ref[...] + p1_ref[...]
    probs = jax.nn.sigmoid(logits)
    eps = 1e-12
    p = jnp.clip(probs, eps, 1.0 - eps)
    pred = pred_ref[...]
    bce = jnp.sum(pred * jnp.log(p) + (1.0 - pred) * jnp.log(1.0 - p))
    m = jax.nn.sigmoid(em_ref[...])
    s_ew = jnp.sum(m)
    ent = -m * jnp.log(m + 1e-15) - (1.0 - m) * jnp.log(1.0 - m + 1e-15)
    s_ent = jnp.sum(ent)
    part = (-bce / (N * C)) + (0.01 / E) * s_ew + (0.5 / E) * s_ent
    acc_ref[0, 0] = acc_ref[0, 0] + part


def _stage3(p0, p1, pred_label, em2d, node_feat_mask):
    return pl.pallas_call(
        _t3_body,
        grid=(NB,),
        in_specs=[
            pl.BlockSpec((BN, C), lambda i: (i, 0)),
            pl.BlockSpec((BN, C), lambda i: (i, 0)),
            pl.BlockSpec((BN, C), lambda i: (i, 0)),
            pl.BlockSpec((1, 1, BE), lambda i: (i, 0, 0)),
            pl.BlockSpec((1, F), lambda i: (0, 0)),
        ],
        out_specs=pl.BlockSpec((1, 1), lambda i: (0, 0),
                               memory_space=pltpu.MemorySpace.SMEM),
        out_shape=jax.ShapeDtypeStruct((1, 1), jnp.float32),
    )(p0, p1, pred_label, em2d, node_feat_mask)


def kernel(x, edge_index, node_feat_mask, edge_mask, W1, W2, pred_label):
    src = edge_index[0]
    dst = edge_index[1]
    em2d = edge_mask.reshape(NB, 1, BE)
    zeros = jnp.zeros((ZLAST, 128), jnp.float32)
    x0 = x[:, :128]
    x1 = x[:, 128:]

    a0, a1 = _sc_segsum((x0, x1), src, dst, edge_mask, zeros,
                        split_edges_by_core=False)

    y = _stage2(a0, a1, node_feat_mask, W1, W2)

    p0, p1 = _sc_segsum((y,), src, dst, edge_mask, zeros,
                        split_edges_by_core=True)

    acc = _stage3(p0, p1, pred_label, em2d, node_feat_mask)
    return acc[0, 0]
